# baseline (speedup 1.0000x reference)
import functools

import jax
import jax.numpy as jnp
from jax import lax
from jax.experimental import pallas as pl
from jax.experimental.pallas import tpu as pltpu
from jax.experimental.pallas import tpu_sc as plsc

B, F, V, D = 16384, 26, 100000, 32
IN_DIM = F * D
PAD_DIM = 896
NCOL = PAD_DIM // 128
EPS = 1e-5
INV = 1.0 / (1.0 + EPS) ** 0.5

NC, NS = 2, 16
NW = NC * NS
N = B * F
BCHUNK = 64
CHUNK = BCHUNK * F
B_PER_W = B // NW
NCHUNK = B_PER_W // BCHUNK


def _gather_body(idx_hbm, table_hbm, out_hbm, idx_v, rows_v, stage_v, sem):
    wid = lax.axis_index("s") * NC + lax.axis_index("c")
    bbase = wid * B_PER_W

    zeros16 = jnp.zeros((16,), jnp.float32)

    def zero_pad(j, carry):
        t, s = j // 8, j % 8
        for k in range(4, 8):
            stage_v[t, NCOL - 1, s, pl.ds(k * 16, 16)] = zeros16
        return carry

    lax.fori_loop(0, 64, zero_pad, 0)

    def permute_row(j, carry):
        t, s = j // 8, j % 8
        rbase = j * F
        for c in range(NCOL - 1):
            for k in range(8):
                f, lane0 = k // 2, (k % 2) * 16
                stage_v[t, c, s, pl.ds(k * 16, 16)] = (
                    rows_v[rbase + 4 * c + f, pl.ds(lane0, 16)]
                )
        for k in range(4):
            f, lane0 = k // 2, (k % 2) * 16
            stage_v[t, NCOL - 1, s, pl.ds(k * 16, 16)] = (
                rows_v[rbase + 24 + f, pl.ds(lane0, 16)]
            )
        return carry

    def step(i, carry):
        off = bbase * F + i * CHUNK
        pltpu.sync_copy(idx_hbm.at[pl.ds(off, CHUNK)], idx_v)
        pltpu.async_copy(table_hbm.at[idx_v], rows_v, sem).wait()
        lax.fori_loop(0, BCHUNK, permute_row, 0)
        pltpu.sync_copy(
            stage_v, out_hbm.at[pl.ds((bbase + i * BCHUNK) // 8, BCHUNK // 8)]
        )
        return carry

    lax.fori_loop(0, NCHUNK, step, 0)


@functools.cache
def _sc_gather():
    return pl.kernel(
        _gather_body,
        out_type=jax.ShapeDtypeStruct((B // 8, NCOL, 8, 128), jnp.float32),
        mesh=plsc.VectorSubcoreMesh(
            core_axis_name="c", subcore_axis_name="s",
            num_cores=NC, num_subcores=NS,
        ),
        scratch_types=[
            pltpu.VMEM((CHUNK,), jnp.int32),
            pltpu.VMEM((CHUNK, D), jnp.float32),
            pltpu.VMEM((BCHUNK // 8, NCOL, 8, 128), jnp.float32),
            pltpu.SemaphoreType.DMA,
        ],
        compiler_params=pltpu.CompilerParams(use_tc_tiling_on_sc=False),
    )


BT = 1024


def _mlp_body(h4_ref, w0, b0, g0, be0, w1, b1, g1, be1, w2, b2, g2, be2, wo,
              bo, out_ref):
    h4 = h4_ref[...]
    w0v = w0[...]
    z = jnp.dot(h4[:, 0].reshape(BT, 128), w0v[0],
                preferred_element_type=jnp.float32)
    for c in range(1, NCOL):
        z = z + jnp.dot(h4[:, c].reshape(BT, 128), w0v[c],
                        preferred_element_type=jnp.float32)
    z = z + b0[...]
    z = jnp.maximum(z, 0.0) * (g0[...] * INV) + be0[...]
    z = jnp.dot(z, w1[...], preferred_element_type=jnp.float32) + b1[...]
    z = jnp.maximum(z, 0.0) * (g1[...] * INV) + be1[...]
    z = jnp.dot(z, w2[...], preferred_element_type=jnp.float32) + b2[...]
    z = jnp.maximum(z, 0.0) * (g2[...] * INV) + be2[...]
    o = jnp.dot(z, wo[...], preferred_element_type=jnp.float32) + bo[...]
    out_ref[...] = jax.nn.sigmoid(o)


def _mlp(h4, W0p, b0, g0, be0, W1T, b1, g1, be1, W2T, b2, g2, be2, WoT, bout):
    full = lambda shape: pl.BlockSpec(shape, lambda i: (0,) * len(shape))
    return pl.pallas_call(
        _mlp_body,
        grid=(B // BT,),
        in_specs=[
            pl.BlockSpec((BT // 8, NCOL, 8, 128), lambda i: (i, 0, 0, 0)),
            full(W0p.shape), full(b0.shape), full(g0.shape), full(be0.shape),
            full(W1T.shape), full(b1.shape), full(g1.shape), full(be1.shape),
            full(W2T.shape), full(b2.shape), full(g2.shape), full(be2.shape),
            full(WoT.shape), full(bout.shape),
        ],
        out_specs=pl.BlockSpec((BT, 1), lambda i: (i, 0)),
        out_shape=jax.ShapeDtypeStruct((B, 1), jnp.float32),
    )(h4, W0p, b0, g0, be0, W1T, b1, g1, be1, W2T, b2, g2, be2, WoT, bout)


def kernel(x, emb_tables, W0, b0, g0, be0, W1, b1, g1, be1, W2, b2, g2, be2,
           Wout, bout):
    flat_idx = (x + jnp.arange(F, dtype=jnp.int32)[None, :] * V).reshape(N)
    table = emb_tables.reshape(F * V, D)
    h4 = _sc_gather()(flat_idx, table)
    W0p = jnp.concatenate(
        [W0.T, jnp.zeros((PAD_DIM - IN_DIM, W0.shape[0]), jnp.float32)], axis=0
    ).reshape(NCOL, 128, W0.shape[0])
    return _mlp(h4, W0p, b0, g0, be0, W1.T, b1, g1, be1, W2.T, b2, g2, be2,
                Wout.T, bout)


# speedup vs baseline: 1.0042x; 1.0042x over previous
import functools

import jax
import jax.numpy as jnp
from jax import lax
from jax.experimental import pallas as pl
from jax.experimental.pallas import tpu as pltpu
from jax.experimental.pallas import tpu_sc as plsc

B, F, V, D = 16384, 26, 100000, 32
IN_DIM = F * D
PAD_DIM = 896
NCOL = PAD_DIM // 128
EPS = 1e-5
INV = 1.0 / (1.0 + EPS) ** 0.5

NC, NS = 2, 16
NW = NC * NS
BCHUNK = 128
B_PER_W = B // NW
NCHUNK = B_PER_W // BCHUNK


def _gather_body(xt_hbm, table_hbm, out_hbm, xblk_v, fslab_v, stage_v, sem):
    wid = lax.axis_index("s") * NC + lax.axis_index("c")
    bbase = wid * B_PER_W

    zeros16 = jnp.zeros((16,), jnp.float32)

    def step(i, carry):
        b0 = bbase + i * BCHUNK
        pltpu.sync_copy(xt_hbm.at[:, pl.ds(b0, BCHUNK)], xblk_v)
        copies = [
            pltpu.async_copy(
                table_hbm.at[f].at[xblk_v.at[f]], fslab_v.at[f], sem
            )
            for f in range(F)
        ]
        for cp in copies:
            cp.wait()

        def tile_row(t, carry2):
            for s in range(8):
                j = 8 * t + s
                for c in range(NCOL - 1):
                    for k in range(8):
                        fld, lane0 = 4 * c + k // 2, (k % 2) * 16
                        stage_v[0, c, s, pl.ds(k * 16, 16)] = (
                            fslab_v[fld, j, pl.ds(lane0, 16)]
                        )
                for k in range(4):
                    fld, lane0 = 24 + k // 2, (k % 2) * 16
                    stage_v[0, NCOL - 1, s, pl.ds(k * 16, 16)] = (
                        fslab_v[fld, j, pl.ds(lane0, 16)]
                    )
                for k in range(4, 8):
                    stage_v[0, NCOL - 1, s, pl.ds(k * 16, 16)] = zeros16
            pltpu.sync_copy(stage_v, out_hbm.at[pl.ds(b0 // 8 + t, 1)])
            return carry2

        lax.fori_loop(0, BCHUNK // 8, tile_row, 0)
        return carry

    lax.fori_loop(0, NCHUNK, step, 0)


@functools.cache
def _sc_gather():
    return pl.kernel(
        _gather_body,
        out_type=jax.ShapeDtypeStruct((B // 8, NCOL, 8, 128), jnp.float32),
        mesh=plsc.VectorSubcoreMesh(
            core_axis_name="c", subcore_axis_name="s",
            num_cores=NC, num_subcores=NS,
        ),
        scratch_types=[
            pltpu.VMEM((F, BCHUNK), jnp.int32),
            pltpu.VMEM((F, BCHUNK, D), jnp.float32),
            pltpu.VMEM((1, NCOL, 8, 128), jnp.float32),
            pltpu.SemaphoreType.DMA,
        ],
        compiler_params=pltpu.CompilerParams(use_tc_tiling_on_sc=False),
    )


BT = 1024


def _mlp_body(h4_ref, w0, b0, g0, be0, w1, b1, g1, be1, w2, b2, g2, be2, wo,
              bo, out_ref):
    h4 = h4_ref[...]
    w0v = w0[...]
    z = jnp.dot(h4[:, 0].reshape(BT, 128), w0v[0],
                preferred_element_type=jnp.float32)
    for c in range(1, NCOL):
        z = z + jnp.dot(h4[:, c].reshape(BT, 128), w0v[c],
                        preferred_element_type=jnp.float32)
    z = z + b0[...]
    z = jnp.maximum(z, 0.0) * (g0[...] * INV) + be0[...]
    z = jnp.dot(z, w1[...], preferred_element_type=jnp.float32) + b1[...]
    z = jnp.maximum(z, 0.0) * (g1[...] * INV) + be1[...]
    z = jnp.dot(z, w2[...], preferred_element_type=jnp.float32) + b2[...]
    z = jnp.maximum(z, 0.0) * (g2[...] * INV) + be2[...]
    o = jnp.dot(z, wo[...], preferred_element_type=jnp.float32) + bo[...]
    out_ref[...] = jax.nn.sigmoid(o)


def _mlp(h4, W0p, b0, g0, be0, W1T, b1, g1, be1, W2T, b2, g2, be2, WoT, bout):
    full = lambda shape: pl.BlockSpec(shape, lambda i: (0,) * len(shape))
    return pl.pallas_call(
        _mlp_body,
        grid=(B // BT,),
        in_specs=[
            pl.BlockSpec((BT // 8, NCOL, 8, 128), lambda i: (i, 0, 0, 0)),
            full(W0p.shape), full(b0.shape), full(g0.shape), full(be0.shape),
            full(W1T.shape), full(b1.shape), full(g1.shape), full(be1.shape),
            full(W2T.shape), full(b2.shape), full(g2.shape), full(be2.shape),
            full(WoT.shape), full(bout.shape),
        ],
        out_specs=pl.BlockSpec((BT, 1), lambda i: (i, 0)),
        out_shape=jax.ShapeDtypeStruct((B, 1), jnp.float32),
    )(h4, W0p, b0, g0, be0, W1T, b1, g1, be1, W2T, b2, g2, be2, WoT, bout)


def kernel(x, emb_tables, W0, b0, g0, be0, W1, b1, g1, be1, W2, b2, g2, be2,
           Wout, bout):
    xT = x.T
    h4 = _sc_gather()(xT, emb_tables)
    W0p = jnp.concatenate(
        [W0.T, jnp.zeros((PAD_DIM - IN_DIM, W0.shape[0]), jnp.float32)], axis=0
    ).reshape(NCOL, 128, W0.shape[0])
    return _mlp(h4, W0p, b0, g0, be0, W1.T, b1, g1, be1, W2.T, b2, g2, be2,
                Wout.T, bout)
